# baseline (speedup 1.0000x reference)
import functools
import jax
import jax.numpy as jnp
from jax import lax
from jax.experimental import pallas as pl
from jax.experimental.pallas import tpu as pltpu
from jax.experimental.pallas import tpu_sc as plsc

B = 4096
L = 200
D = 64
TOTAL = B * L

_info = plsc.get_sparse_core_info()
NC = _info.num_cores
NS = _info.num_subcores
NW = NC * NS
T_QUADS = L // 4

_mesh = plsc.VectorSubcoreMesh(core_axis_name="c", subcore_axis_name="s")


@functools.partial(
    pl.kernel,
    mesh=_mesh,
    out_type=jax.ShapeDtypeStruct((L, 8, 32, 8, 128), jnp.float32),
    scratch_types=[
        pltpu.VMEM((2, 128), jnp.int32),
        pltpu.VMEM((2, 128), jnp.int32),
        pltpu.VMEM((128, D), jnp.float32),
        pltpu.VMEM((128, D), jnp.float32),
        pltpu.VMEM((8, 8, 129), jnp.float32),
        pltpu.VMEM((8, 8, 129), jnp.float32),
        pltpu.SemaphoreType.DMA,
        pltpu.SemaphoreType.DMA,
        pltpu.SemaphoreType.DMA,
        pltpu.SemaphoreType.DMA,
        pltpu.SemaphoreType.DMA,
    ],
    compiler_params=pltpu.CompilerParams(use_tc_tiling_on_sc=False,
                                         needs_layout_passes=False),
)
def _gather_kernel(idx5_hbm, u_hbm, out_hbm,
                   idx_x, idx_y, g_a, g_b, o_a, o_b,
                   isem, gsem_a, gsem_b, wsem_a, wsem_b):
    bc = lax.axis_index("s") * NC + lax.axis_index("c")

    iot = lax.iota(jnp.int32, 16)
    frv = [(iot + 16 * c) >> 3 for c in range(4)]
    sv = [(iot + 16 * c) & 7 for c in range(4)]

    def idx_at(l):
        return idx5_hbm.at[l // 8, bc, l % 8]

    def out_at(l):
        return out_hbm.at[l, :, bc]

    def o_slice(o_v):
        return o_v.at[:, :, pl.ds(0, 128)]

    def drain_idx(l, dst):
        pltpu.make_async_copy(idx_at(l), dst, isem).wait()

    def shuffle(g_v, o_v):
        def blk(bb, blv):
            for u8 in range(8):
                for c in range(4):
                    vals = g_v[bb * 8 + u8, pl.ds(16 * c, 16)]
                    plsc.store_scatter(o_v, [frv[c], sv[c], blv], vals)
                blv = blv + 1
            return blv
        lax.fori_loop(0, 16, blk, iot * 0)

    pltpu.async_copy(idx_at(0), idx_x.at[0], isem)
    pltpu.async_copy(idx_at(1), idx_x.at[1], isem)
    drain_idx(0, idx_x.at[0])
    drain_idx(1, idx_x.at[1])
    pltpu.async_copy(u_hbm.at[idx_x.at[0]], g_a, gsem_a)

    def body(t, carry):
        l0 = 4 * t
        l1 = l0 + 1
        l2 = l0 + 2
        l3 = l0 + 3

        pltpu.async_copy(u_hbm.at[idx_x.at[1]], g_b, gsem_b)
        pltpu.async_copy(idx_at(l2), idx_y.at[0], isem)
        pltpu.async_copy(idx_at(l3), idx_y.at[1], isem)

        pltpu.make_async_copy(u_hbm.at[idx_x.at[0]], g_a, gsem_a).wait()

        @pl.when(t > 0)
        def _():
            pltpu.make_async_copy(o_slice(o_a), out_at(l0), wsem_a).wait()

        shuffle(g_a, o_a)
        pltpu.async_copy(o_slice(o_a), out_at(l0), wsem_a)

        pltpu.make_async_copy(u_hbm.at[idx_x.at[1]], g_b, gsem_b).wait()

        @pl.when(t > 0)
        def _():
            pltpu.make_async_copy(o_slice(o_b), out_at(l1), wsem_b).wait()

        shuffle(g_b, o_b)
        pltpu.async_copy(o_slice(o_b), out_at(l1), wsem_b)

        drain_idx(l2, idx_y.at[0])
        drain_idx(l3, idx_y.at[1])
        pltpu.async_copy(u_hbm.at[idx_y.at[0]], g_a, gsem_a)
        pltpu.async_copy(u_hbm.at[idx_y.at[1]], g_b, gsem_b)

        @pl.when(t < T_QUADS - 1)
        def _():
            pltpu.async_copy(idx_at(l0 + 4), idx_x.at[0], isem)
            pltpu.async_copy(idx_at(l1 + 4), idx_x.at[1], isem)

        pltpu.make_async_copy(u_hbm.at[idx_y.at[0]], g_a, gsem_a).wait()
        pltpu.make_async_copy(o_slice(o_a), out_at(l2), wsem_a).wait()
        shuffle(g_a, o_a)
        pltpu.async_copy(o_slice(o_a), out_at(l2), wsem_a)

        pltpu.make_async_copy(u_hbm.at[idx_y.at[1]], g_b, gsem_b).wait()
        pltpu.make_async_copy(o_slice(o_b), out_at(l3), wsem_b).wait()
        shuffle(g_b, o_b)
        pltpu.async_copy(o_slice(o_b), out_at(l3), wsem_b)

        @pl.when(t < T_QUADS - 1)
        def _():
            drain_idx(l0 + 4, idx_x.at[0])
            drain_idx(l1 + 4, idx_x.at[1])
            pltpu.async_copy(u_hbm.at[idx_x.at[0]], g_a, gsem_a)

        return carry

    lax.fori_loop(0, T_QUADS, body, 0)
    pltpu.make_async_copy(o_slice(o_a), out_at(L - 2), wsem_a).wait()
    pltpu.make_async_copy(o_slice(o_b), out_at(L - 1), wsem_b).wait()


def _mask_body(idx_ref, out_ref):
    out_ref[...] = (idx_ref[...] > 0).astype(jnp.int32)


_mask = pl.pallas_call(
    _mask_body,
    out_shape=jax.ShapeDtypeStruct((6400, 128), jnp.int32),
    grid=(8,),
    in_specs=[pl.BlockSpec((800, 128), lambda i: (i, 0))],
    out_specs=pl.BlockSpec((800, 128), lambda i: (i, 0)),
)


@jax.jit
def kernel(word_idx, tune_pre_trained, table):
    wi = word_idx.astype(jnp.int32)
    idx5 = wi.T.reshape(25, 8, 32, 128).transpose(0, 2, 1, 3)
    ol = _gather_kernel(idx5, table)
    static_emb = ol.transpose(2, 4, 0, 1, 3).reshape(B, L, D)
    mask = _mask(wi.reshape(6400, 128))
    bottom_existence = mask.reshape(B, L, 1).astype(jnp.bool_)
    return (static_emb, bottom_existence)
